# baseline (speedup 1.0000x reference)
import functools

import jax
import jax.numpy as jnp
from jax import lax
from jax.experimental import pallas as pl
from jax.experimental.pallas import tpu as pltpu
from jax.experimental.pallas import tpu_sc as plsc

NC = 2
NS = 16
NW = NC * NS
CHUNK = 128
DEGW = 128
DEGQ = 8

_mesh = plsc.VectorSubcoreMesh(core_axis_name="c", subcore_axis_name="s")


def _make_deg_kernel(npad, ch):
    rows_per_tile = npad // NS

    @functools.partial(
        pl.kernel,
        out_type=jax.ShapeDtypeStruct((NC, npad, DEGW), jnp.float32),
        mesh=_mesh,
        scratch_types=[
            pltpu.VMEM((ch, CHUNK), jnp.int32),
            pltpu.VMEM((CHUNK, DEGW), jnp.float32),
            pltpu.VMEM_SHARED((npad, DEGW), jnp.float32),
            pltpu.SemaphoreType.DMA((DEGQ,)),
        ],
    )
    def deg_kernel(dst_hbm, ones_hbm, zdeg_hbm, out_hbm, idx_v, ones_v, acc_sh,
                   ssem):
        c = lax.axis_index("c")
        s = lax.axis_index("s")
        pltpu.sync_copy(ones_hbm, ones_v)
        pltpu.sync_copy(zdeg_hbm, acc_sh.at[pl.ds(s * rows_per_tile, rows_per_tile)])
        plsc.subcore_barrier()

        pltpu.sync_copy(dst_hbm.at[c, s], idx_v)

        def body(j, _):
            q = lax.rem(j, DEGQ)

            @pl.when(j >= DEGQ)
            def _():
                pltpu.make_async_copy(ones_hbm, ones_v, ssem.at[q]).wait()

            pltpu.async_copy(ones_v, acc_sh.at[idx_v.at[j]], ssem.at[q], add=True)
            return 0

        lax.fori_loop(0, ch, body, 0)

        def drain(j, _):
            q = lax.rem(j, DEGQ)
            pltpu.make_async_copy(ones_hbm, ones_v, ssem.at[q]).wait()
            return 0

        lax.fori_loop(ch - DEGQ, ch, drain, 0)
        plsc.subcore_barrier()
        pltpu.sync_copy(
            acc_sh.at[pl.ds(s * rows_per_tile, rows_per_tile)],
            out_hbm.at[c, pl.ds(s * rows_per_tile, rows_per_tile)],
        )

    return deg_kernel


NBUF = 2
KBLK = 16


def _make_scatter_kernel(npad, d, ch):
    rows_per_tile = npad // NS
    nb = ch // KBLK

    @functools.partial(
        pl.kernel,
        out_type=jax.ShapeDtypeStruct((NC, npad, d), jnp.float32),
        mesh=_mesh,
        scratch_types=[
            pltpu.VMEM((KBLK, 2, CHUNK), jnp.int32),
            pltpu.VMEM((NBUF, CHUNK, d), jnp.float32),
            pltpu.VMEM_SHARED((npad, d), jnp.float32),
        ],
    )
    def scat_kernel(hs_hbm, ei_hbm, zrows_hbm, out_hbm, blk_v, rows_v, acc_sh):
        c = lax.axis_index("c")
        s = lax.axis_index("s")
        pltpu.sync_copy(zrows_hbm, acc_sh.at[pl.ds(s * rows_per_tile, rows_per_tile)])
        plsc.subcore_barrier()

        def outer(m, _):
            pltpu.sync_copy(ei_hbm.at[c, s, pl.ds(m * KBLK, KBLK)], blk_v)

            def wave(w, _):
                @plsc.parallel_loop(0, NBUF)
                def _wave(i):
                    t = w * NBUF + i
                    pltpu.sync_copy(hs_hbm.at[blk_v.at[t, 0]], rows_v.at[i])
                    pltpu.sync_copy(rows_v.at[i], acc_sh.at[blk_v.at[t, 1]],
                                    add=True)

                return 0

            lax.fori_loop(0, KBLK // NBUF, wave, 0)
            return 0

        lax.fori_loop(0, nb, outer, 0)
        plsc.subcore_barrier()
        pltpu.sync_copy(
            acc_sh.at[pl.ds(s * rows_per_tile, rows_per_tile)],
            out_hbm.at[c, pl.ds(s * rows_per_tile, rows_per_tile)],
        )

    return scat_kernel


def _dinv(d0_ref, d1_ref):
    deg = 1.0 + d0_ref[:, 0:1] + d1_ref[:, 0:1]
    return lax.rsqrt(deg)


def _tc1_body(x_ref, w_ref, d0_ref, d1_ref, hs_ref):
    h = jnp.dot(x_ref[:], w_ref[:], preferred_element_type=jnp.float32)
    hs_ref[:] = h * _dinv(d0_ref, d1_ref)


def _tc2_body(a0_ref, a1_ref, hs_ref, d0_ref, d1_ref, b_ref, w_ref, out_ref):
    dinv = _dinv(d0_ref, d1_ref)
    h1 = dinv * (a0_ref[:] + a1_ref[:] + hs_ref[:]) + b_ref[:]
    h1 = jnp.maximum(h1, 0.0)
    out_ref[:] = jnp.dot(h1, w_ref[:], preferred_element_type=jnp.float32) * dinv


def _tc3_body(a0_ref, a1_ref, hs_ref, d0_ref, d1_ref, b_ref, out_ref):
    dinv = _dinv(d0_ref, d1_ref)
    out_ref[:] = dinv * (a0_ref[:] + a1_ref[:] + hs_ref[:]) + b_ref[:]


def _row_spec(blk, d):
    return pl.BlockSpec((blk, d), lambda i: (i, 0))


def _full_spec(shape):
    return pl.BlockSpec(shape, lambda i: tuple(0 for _ in shape))


def kernel(x, edge_index, W1, b1, W2, b2):
    n, d_in = x.shape
    d_hid = W1.shape[1]
    d_out = W2.shape[1]
    e = edge_index.shape[1]

    npad = ((n + 1024 - 1) // 1024) * 1024
    blk = npad
    ch = (e + NW * CHUNK - 1) // (NW * CHUNK)
    ch = ((ch + KBLK - 1) // KBLK) * KBLK
    epad = NW * ch * CHUNK

    src = edge_index[0].astype(jnp.int32)
    dst = edge_index[1].astype(jnp.int32)
    pad_idx = jnp.full((epad - e,), n, dtype=jnp.int32)
    src_p = jnp.concatenate([src, pad_idx]).reshape(NC, NS, ch, CHUNK)
    dst_p = jnp.concatenate([dst, pad_idx]).reshape(NC, NS, ch, CHUNK)
    ei_p = jnp.stack([src_p, dst_p], axis=3)
    x_p = jnp.concatenate([x, jnp.zeros((npad - n, d_in), x.dtype)], axis=0)

    rows_per_tile = npad // NS
    ones_rows = jnp.ones((CHUNK, DEGW), jnp.float32)
    zrows = jnp.zeros((rows_per_tile, d_hid), jnp.float32)

    deg2 = _make_deg_kernel(npad, ch)(dst_p, ones_rows, zrows)
    d0, d1 = deg2[0], deg2[1]

    grid = npad // blk
    hs1 = pl.pallas_call(
        _tc1_body,
        grid=(grid,),
        in_specs=[
            _row_spec(blk, d_in),
            _full_spec((d_in, d_hid)),
            _row_spec(blk, DEGW),
            _row_spec(blk, DEGW),
        ],
        out_specs=_row_spec(blk, d_hid),
        out_shape=jax.ShapeDtypeStruct((npad, d_hid), jnp.float32),
    )(x_p, W1, d0, d1)

    scat = _make_scatter_kernel(npad, d_hid, ch)
    acc1 = scat(hs1, ei_p, zrows)

    hs2 = pl.pallas_call(
        _tc2_body,
        grid=(grid,),
        in_specs=[
            _row_spec(blk, d_hid),
            _row_spec(blk, d_hid),
            _row_spec(blk, d_hid),
            _row_spec(blk, DEGW),
            _row_spec(blk, DEGW),
            _full_spec((1, d_hid)),
            _full_spec((d_hid, d_out)),
        ],
        out_specs=_row_spec(blk, d_out),
        out_shape=jax.ShapeDtypeStruct((npad, d_out), jnp.float32),
    )(acc1[0], acc1[1], hs1, d0, d1, b1.reshape(1, d_hid), W2)

    acc2 = scat(hs2, ei_p, zrows)

    out = pl.pallas_call(
        _tc3_body,
        grid=(grid,),
        in_specs=[
            _row_spec(blk, d_out),
            _row_spec(blk, d_out),
            _row_spec(blk, d_out),
            _row_spec(blk, DEGW),
            _row_spec(blk, DEGW),
            _full_spec((1, d_out)),
        ],
        out_specs=_row_spec(blk, d_out),
        out_shape=jax.ShapeDtypeStruct((npad, d_out), jnp.float32),
    )(acc2[0], acc2[1], hs2, d0, d1, b2.reshape(1, d_out))

    return out[:n]


# speedup vs baseline: 1.5677x; 1.5677x over previous
import functools

import jax
import jax.numpy as jnp
from jax import lax
from jax.experimental import pallas as pl
from jax.experimental.pallas import tpu as pltpu
from jax.experimental.pallas import tpu_sc as plsc

NC = 2
NS = 16
NW = NC * NS
CHUNK = 128
DEGW = 128
DEGQ = 8

_mesh = plsc.VectorSubcoreMesh(core_axis_name="c", subcore_axis_name="s")


def _make_deg_kernel(npad, ch):
    rows_per_tile = npad // NS

    @functools.partial(
        pl.kernel,
        out_type=jax.ShapeDtypeStruct((NC, npad, DEGW), jnp.float32),
        mesh=_mesh,
        scratch_types=[
            pltpu.VMEM((ch, CHUNK), jnp.int32),
            pltpu.VMEM((CHUNK, DEGW), jnp.float32),
            pltpu.VMEM_SHARED((npad, DEGW), jnp.float32),
            pltpu.SemaphoreType.DMA((DEGQ,)),
        ],
    )
    def deg_kernel(dst_hbm, ones_hbm, zdeg_hbm, out_hbm, idx_v, ones_v, acc_sh,
                   ssem):
        c = lax.axis_index("c")
        s = lax.axis_index("s")
        pltpu.sync_copy(ones_hbm, ones_v)
        pltpu.sync_copy(zdeg_hbm, acc_sh.at[pl.ds(s * rows_per_tile, rows_per_tile)])
        plsc.subcore_barrier()

        pltpu.sync_copy(dst_hbm.at[c, s], idx_v)

        def body(j, _):
            q = lax.rem(j, DEGQ)

            @pl.when(j >= DEGQ)
            def _():
                pltpu.make_async_copy(ones_hbm, ones_v, ssem.at[q]).wait()

            pltpu.async_copy(ones_v, acc_sh.at[idx_v.at[j]], ssem.at[q], add=True)
            return 0

        lax.fori_loop(0, ch, body, 0)

        def drain(j, _):
            q = lax.rem(j, DEGQ)
            pltpu.make_async_copy(ones_hbm, ones_v, ssem.at[q]).wait()
            return 0

        lax.fori_loop(ch - DEGQ, ch, drain, 0)
        plsc.subcore_barrier()
        pltpu.sync_copy(
            acc_sh.at[pl.ds(s * rows_per_tile, rows_per_tile)],
            out_hbm.at[c, pl.ds(s * rows_per_tile, rows_per_tile)],
        )

    return deg_kernel


def _make_scatter_kernel(npad, d, ch):
    rows_per_tile = npad // NS

    @functools.partial(
        pl.kernel,
        out_type=jax.ShapeDtypeStruct((NC, npad, d), jnp.float32),
        mesh=_mesh,
        scratch_types=[
            pltpu.VMEM((ch, 2, CHUNK), jnp.int32),
            pltpu.VMEM((CHUNK, d), jnp.float32),
            pltpu.VMEM_SHARED((npad, d), jnp.float32),
        ],
    )
    def scat_kernel(hs_hbm, ei_hbm, zrows_hbm, out_hbm, idx_v, rows_v, acc_sh):
        c = lax.axis_index("c")
        s = lax.axis_index("s")
        pltpu.sync_copy(zrows_hbm, acc_sh.at[pl.ds(s * rows_per_tile, rows_per_tile)])
        plsc.subcore_barrier()
        pltpu.sync_copy(ei_hbm.at[c, s], idx_v)

        def body(j, _):
            pltpu.sync_copy(hs_hbm.at[idx_v.at[j, 0]], rows_v)
            pltpu.sync_copy(rows_v, acc_sh.at[idx_v.at[j, 1]], add=True)
            return 0

        lax.fori_loop(0, ch, body, 0)
        plsc.subcore_barrier()
        pltpu.sync_copy(
            acc_sh.at[pl.ds(s * rows_per_tile, rows_per_tile)],
            out_hbm.at[c, pl.ds(s * rows_per_tile, rows_per_tile)],
        )

    return scat_kernel


def _dinv(d0_ref, d1_ref):
    deg = 1.0 + d0_ref[:, 0:1] + d1_ref[:, 0:1]
    return lax.rsqrt(deg)


def _tc1_body(x_ref, w_ref, d0_ref, d1_ref, hs_ref):
    h = jnp.dot(x_ref[:], w_ref[:], preferred_element_type=jnp.float32)
    hs_ref[:] = h * _dinv(d0_ref, d1_ref)


def _tc2_body(a0_ref, a1_ref, hs_ref, d0_ref, d1_ref, b_ref, w_ref, out_ref):
    dinv = _dinv(d0_ref, d1_ref)
    h1 = dinv * (a0_ref[:] + a1_ref[:] + hs_ref[:]) + b_ref[:]
    h1 = jnp.maximum(h1, 0.0)
    out_ref[:] = jnp.dot(h1, w_ref[:], preferred_element_type=jnp.float32) * dinv


def _tc3_body(a0_ref, a1_ref, hs_ref, d0_ref, d1_ref, b_ref, out_ref):
    dinv = _dinv(d0_ref, d1_ref)
    out_ref[:] = dinv * (a0_ref[:] + a1_ref[:] + hs_ref[:]) + b_ref[:]


def _row_spec(blk, d):
    return pl.BlockSpec((blk, d), lambda i: (i, 0))


def _full_spec(shape):
    return pl.BlockSpec(shape, lambda i: tuple(0 for _ in shape))


def kernel(x, edge_index, W1, b1, W2, b2):
    n, d_in = x.shape
    d_hid = W1.shape[1]
    d_out = W2.shape[1]
    e = edge_index.shape[1]

    npad = ((n + 1024 - 1) // 1024) * 1024
    blk = npad
    ch = (e + NW * CHUNK - 1) // (NW * CHUNK)
    epad = NW * ch * CHUNK

    src = edge_index[0].astype(jnp.int32)
    dst = edge_index[1].astype(jnp.int32)
    pad_idx = jnp.full((epad - e,), n, dtype=jnp.int32)
    src_p = jnp.concatenate([src, pad_idx]).reshape(NC, NS, ch, CHUNK)
    dst_p = jnp.concatenate([dst, pad_idx]).reshape(NC, NS, ch, CHUNK)
    ei_p = jnp.stack([src_p, dst_p], axis=3)
    x_p = jnp.concatenate([x, jnp.zeros((npad - n, d_in), x.dtype)], axis=0)

    rows_per_tile = npad // NS
    ones_rows = jnp.ones((CHUNK, DEGW), jnp.float32)
    zrows = jnp.zeros((rows_per_tile, d_hid), jnp.float32)

    deg2 = _make_deg_kernel(npad, ch)(dst_p, ones_rows, zrows)
    d0, d1 = deg2[0], deg2[1]

    grid = npad // blk
    hs1 = pl.pallas_call(
        _tc1_body,
        grid=(grid,),
        in_specs=[
            _row_spec(blk, d_in),
            _full_spec((d_in, d_hid)),
            _row_spec(blk, DEGW),
            _row_spec(blk, DEGW),
        ],
        out_specs=_row_spec(blk, d_hid),
        out_shape=jax.ShapeDtypeStruct((npad, d_hid), jnp.float32),
    )(x_p, W1, d0, d1)

    scat = _make_scatter_kernel(npad, d_hid, ch)
    acc1 = scat(hs1, ei_p, zrows)

    hs2 = pl.pallas_call(
        _tc2_body,
        grid=(grid,),
        in_specs=[
            _row_spec(blk, d_hid),
            _row_spec(blk, d_hid),
            _row_spec(blk, d_hid),
            _row_spec(blk, DEGW),
            _row_spec(blk, DEGW),
            _full_spec((1, d_hid)),
            _full_spec((d_hid, d_out)),
        ],
        out_specs=_row_spec(blk, d_out),
        out_shape=jax.ShapeDtypeStruct((npad, d_out), jnp.float32),
    )(acc1[0], acc1[1], hs1, d0, d1, b1.reshape(1, d_hid), W2)

    acc2 = scat(hs2, ei_p, zrows)

    out = pl.pallas_call(
        _tc3_body,
        grid=(grid,),
        in_specs=[
            _row_spec(blk, d_out),
            _row_spec(blk, d_out),
            _row_spec(blk, d_out),
            _row_spec(blk, DEGW),
            _row_spec(blk, DEGW),
            _full_spec((1, d_out)),
        ],
        out_specs=_row_spec(blk, d_out),
        out_shape=jax.ShapeDtypeStruct((npad, d_out), jnp.float32),
    )(acc2[0], acc2[1], hs2, d0, d1, b2.reshape(1, d_out))

    return out[:n]
